# Initial kernel scaffold; baseline (speedup 1.0000x reference)
import jax
import jax.numpy as jnp
from jax.experimental import pallas as pl


def kernel(node_feat, memory, root_nid, neighbor_nid, root_ts, neighbor_ts, neighbor_edge_feature, Wq, Wk, Wv, Wo, Wsrc, Wdst, Wout):
    raise NotImplementedError("write your pallas kernel here")


# speedup vs baseline: 3.9417x; 3.9417x over previous
import functools

import jax
import jax.numpy as jnp
import numpy as np
from jax import lax
from jax.experimental import pallas as pl
from jax.experimental.pallas import tpu as pltpu
from jax.experimental.pallas import tpu_sc as plsc

N_NODES = 100000
B = 4096
N_ROOT = 3 * B
K = 32
E = N_ROOT * K
D_NODE = 128
D_EDGE = 16
PW = 128
NW = 32
E_PER_W = E // NW
EC = 128
N_ECHUNK = E_PER_W // EC
R_PER_W = N_ROOT // NW
RC = 128
N_RCHUNK = R_PER_W // RC
NB = 1000
RB = 128
SCALE = 1.0 / np.sqrt(50.0)


def _pad2(w, r, c):
    return jnp.zeros((r, c), jnp.float32).at[: w.shape[0], : w.shape[1]].set(w)


def _precompute_body(nf, mem, wkf, wkm, wvf, wvm, wqf, wqm, qb, tk, tv, tq):
    nfb = nf[...]
    memb = mem[...]
    tk[...] = jnp.dot(nfb, wkf[...], preferred_element_type=jnp.float32) + \
              jnp.dot(memb, wkm[...], preferred_element_type=jnp.float32)
    tv[...] = jnp.dot(nfb, wvf[...], preferred_element_type=jnp.float32) + \
              jnp.dot(memb, wvm[...], preferred_element_type=jnp.float32)
    tq[...] = jnp.dot(nfb, wqf[...], preferred_element_type=jnp.float32) + \
              jnp.dot(memb, wqm[...], preferred_element_type=jnp.float32) + qb[...]


def _precompute(node_feat, memory, wkf, wkm, wvf, wvm, wqf, wqm, qb):
    nblk = N_NODES // NB
    w_spec = lambda shp: pl.BlockSpec(shp, lambda i: (0, 0))
    return pl.pallas_call(
        _precompute_body,
        grid=(nblk,),
        in_specs=[
            pl.BlockSpec((NB, D_NODE), lambda i: (i, 0)),
            pl.BlockSpec((NB, 100), lambda i: (i, 0)),
            w_spec((D_NODE, PW)), w_spec((100, PW)),
            w_spec((D_NODE, PW)), w_spec((100, PW)),
            w_spec((D_NODE, PW)), w_spec((100, PW)),
            w_spec((1, PW)),
        ],
        out_specs=[
            pl.BlockSpec((NB, PW), lambda i: (i, 0)),
            pl.BlockSpec((NB, PW), lambda i: (i, 0)),
            pl.BlockSpec((NB, PW), lambda i: (i, 0)),
        ],
        out_shape=[
            jax.ShapeDtypeStruct((N_NODES, PW), jnp.float32),
            jax.ShapeDtypeStruct((N_NODES, PW), jnp.float32),
            jax.ShapeDtypeStruct((N_NODES, PW), jnp.float32),
        ],
    )(node_feat, memory, wkf, wkm, wvf, wvm, wqf, wqm, qb)


def _sc_gather(tk, tv, tq, node_feat, flat_nb, root_nid):
    mesh = plsc.VectorSubcoreMesh(core_axis_name="c", subcore_axis_name="s")

    @functools.partial(
        pl.kernel,
        mesh=mesh,
        out_type=[
            jax.ShapeDtypeStruct((E, PW), jnp.float32),
            jax.ShapeDtypeStruct((E, PW), jnp.float32),
            jax.ShapeDtypeStruct((N_ROOT, PW), jnp.float32),
            jax.ShapeDtypeStruct((N_ROOT, D_NODE), jnp.float32),
        ],
        scratch_types=[
            pltpu.VMEM((EC,), jnp.int32),
            pltpu.VMEM((EC, PW), jnp.float32),
            pltpu.VMEM((EC, PW), jnp.float32),
            pltpu.VMEM((RC,), jnp.int32),
            pltpu.VMEM((RC, PW), jnp.float32),
            pltpu.VMEM((RC, D_NODE), jnp.float32),
            pltpu.SemaphoreType.DMA,
            pltpu.SemaphoreType.DMA,
        ],
    )
    def gather_k(tk_h, tv_h, tq_h, nf_h, nbid_h, rid_h, gk_h, gv_h, qr_h, fr_h,
                 idx_v, rk_v, rv_v, idx2_v, rq_v, rf_v, sem_a, sem_b):
        wid = lax.axis_index("s") * 2 + lax.axis_index("c")
        ebase = wid * E_PER_W

        def edge_body(i, carry):
            off = ebase + i * EC
            pltpu.sync_copy(nbid_h.at[pl.ds(off, EC)], idx_v)
            cp_a = pltpu.async_copy(tk_h.at[idx_v], rk_v, sem_a)
            cp_b = pltpu.async_copy(tv_h.at[idx_v], rv_v, sem_b)
            cp_a.wait()
            cp_b.wait()
            pltpu.sync_copy(rk_v, gk_h.at[pl.ds(off, EC)])
            pltpu.sync_copy(rv_v, gv_h.at[pl.ds(off, EC)])
            return carry

        lax.fori_loop(0, N_ECHUNK, edge_body, 0)

        rbase = wid * R_PER_W

        def root_body(i, carry):
            off = rbase + i * RC
            pltpu.sync_copy(rid_h.at[pl.ds(off, RC)], idx2_v)
            cp_a = pltpu.async_copy(tq_h.at[idx2_v], rq_v, sem_a)
            cp_b = pltpu.async_copy(nf_h.at[idx2_v], rf_v, sem_b)
            cp_a.wait()
            cp_b.wait()
            pltpu.sync_copy(rq_v, qr_h.at[pl.ds(off, RC)])
            pltpu.sync_copy(rf_v, fr_h.at[pl.ds(off, RC)])
            return carry

        lax.fori_loop(0, N_RCHUNK, root_body, 0)

    return gather_k(tk, tv, tq, node_feat, flat_nb, root_nid)


def _attn_body(gk, gv, qr, fr, rts, nts, ef, wrow, wke, wkt, wve, wvt, wof,
               woa, h_out):
    f32 = jnp.float32
    et = jnp.cos((rts[...] - nts[...]) * wrow[...])
    kb = gk[...] + jnp.dot(et, wkt[...], preferred_element_type=f32) + \
         jnp.dot(ef[...], wke[...], preferred_element_type=f32)
    vb = gv[...] + jnp.dot(et, wvt[...], preferred_element_type=f32) + \
         jnp.dot(ef[...], wve[...], preferred_element_type=f32)
    k3 = kb.reshape(RB, K, PW)
    v3 = vb.reshape(RB, K, PW)
    q = qr[...]
    p3 = k3 * q[:, None, :]
    lane = lax.broadcasted_iota(jnp.int32, (1, 1, PW), 2)
    m_a = (lane < 50).astype(f32)
    m_b = ((lane >= 50) & (lane < 100)).astype(f32)
    s_a = jnp.clip(jnp.sum(p3 * m_a, axis=2) * SCALE, -10.0, 10.0)
    s_b = jnp.clip(jnp.sum(p3 * m_b, axis=2) * SCALE, -10.0, 10.0)
    e_a = jnp.exp(s_a - jnp.max(s_a, axis=1, keepdims=True))
    e_b = jnp.exp(s_b - jnp.max(s_b, axis=1, keepdims=True))
    a_a = e_a / jnp.sum(e_a, axis=1, keepdims=True)
    a_b = e_b / jnp.sum(e_b, axis=1, keepdims=True)
    w3 = a_a[:, :, None] * m_a + a_b[:, :, None] * m_b
    agg = jnp.sum(w3 * v3, axis=1)
    h = jnp.maximum(
        jnp.dot(fr[...], wof[...], preferred_element_type=f32) +
        jnp.dot(agg, woa[...], preferred_element_type=f32), 0.0)
    h_out[...] = h


def _attention(gk, gv, qr, fr, rts_e, nts_e, ef, wrow, wke, wkt, wve, wvt,
               wof, woa):
    nblk = N_ROOT // RB
    w_spec = lambda shp: pl.BlockSpec(shp, lambda i: (0, 0))
    return pl.pallas_call(
        _attn_body,
        grid=(nblk,),
        in_specs=[
            pl.BlockSpec((RB * K, PW), lambda i: (i, 0)),
            pl.BlockSpec((RB * K, PW), lambda i: (i, 0)),
            pl.BlockSpec((RB, PW), lambda i: (i, 0)),
            pl.BlockSpec((RB, D_NODE), lambda i: (i, 0)),
            pl.BlockSpec((RB * K, 1), lambda i: (i, 0)),
            pl.BlockSpec((RB * K, 1), lambda i: (i, 0)),
            pl.BlockSpec((RB * K, D_EDGE), lambda i: (i, 0)),
            w_spec((1, PW)),
            w_spec((D_EDGE, PW)), w_spec((PW, PW)),
            w_spec((D_EDGE, PW)), w_spec((PW, PW)),
            w_spec((D_NODE, PW)), w_spec((PW, PW)),
        ],
        out_specs=pl.BlockSpec((RB, PW), lambda i: (i, 0)),
        out_shape=jax.ShapeDtypeStruct((N_ROOT, PW), jnp.float32),
    )(gk, gv, qr, fr, rts_e, nts_e, ef, wrow, wke, wkt, wve, wvt, wof, woa)


def _pred_body(hs, hp, hn, wsrc, wdst, wout, pos, neg):
    f32 = jnp.float32
    h_src = jnp.dot(hs[...], wsrc[...], preferred_element_type=f32)
    h_pos = jnp.dot(hp[...], wdst[...], preferred_element_type=f32)
    h_neg = jnp.dot(hn[...], wdst[...], preferred_element_type=f32)
    pos[...] = jnp.dot(jnp.maximum(h_src + h_pos, 0.0), wout[...],
                       preferred_element_type=f32)
    neg[...] = jnp.dot(jnp.maximum(h_src + h_neg, 0.0), wout[...],
                       preferred_element_type=f32)


def _predict(h_src, h_pos, h_neg, wsrc, wdst, wout):
    PB = 512
    nblk = B // PB
    w_spec = lambda shp: pl.BlockSpec(shp, lambda i: (0, 0))
    return pl.pallas_call(
        _pred_body,
        grid=(nblk,),
        in_specs=[
            pl.BlockSpec((PB, PW), lambda i: (i, 0)),
            pl.BlockSpec((PB, PW), lambda i: (i, 0)),
            pl.BlockSpec((PB, PW), lambda i: (i, 0)),
            w_spec((PW, PW)), w_spec((PW, PW)), w_spec((PW, 8)),
        ],
        out_specs=[
            pl.BlockSpec((PB, 8), lambda i: (i, 0)),
            pl.BlockSpec((PB, 8), lambda i: (i, 0)),
        ],
        out_shape=[
            jax.ShapeDtypeStruct((B, 8), jnp.float32),
            jax.ShapeDtypeStruct((B, 8), jnp.float32),
        ],
    )(h_src, h_pos, h_neg, wsrc, wdst, wout)


def kernel(node_feat, memory, root_nid, neighbor_nid, root_ts, neighbor_ts,
           neighbor_edge_feature, Wq, Wk, Wv, Wo, Wsrc, Wdst, Wout):
    f32 = jnp.float32
    wqf = _pad2(Wq[:128], D_NODE, PW)
    wqm = _pad2(Wq[228:], 100, PW)
    qb = _pad2(jnp.sum(Wq[128:228], 0, keepdims=True), 1, PW)
    wkf = _pad2(Wk[:128], D_NODE, PW)
    wke = _pad2(Wk[128:144], D_EDGE, PW)
    wkt = _pad2(Wk[144:244], PW, PW)
    wkm = _pad2(Wk[244:], 100, PW)
    wvf = _pad2(Wv[:128], D_NODE, PW)
    wve = _pad2(Wv[128:144], D_EDGE, PW)
    wvt = _pad2(Wv[144:244], PW, PW)
    wvm = _pad2(Wv[244:], 100, PW)
    wof = _pad2(Wo[:128], D_NODE, PW)
    woa = _pad2(Wo[128:], PW, PW)
    wsrc = _pad2(Wsrc, PW, PW)
    wdst = _pad2(Wdst, PW, PW)
    wout = _pad2(Wout, PW, 8)
    wrow = _pad2((1.0 / (10.0 ** jnp.linspace(0.0, 9.0, 100, dtype=f32)))[None, :], 1, PW)

    tk, tv, tq = _precompute(node_feat, memory, wkf, wkm, wvf, wvm, wqf, wqm, qb)

    flat_nb = neighbor_nid.reshape(-1).astype(jnp.int32)
    rid = root_nid.astype(jnp.int32)
    gk, gv, qr, fr = _sc_gather(tk, tv, tq, node_feat, flat_nb, rid)

    rts_e = jnp.broadcast_to(root_ts[:, None], (N_ROOT, K)).reshape(E, 1)
    nts_e = neighbor_ts.reshape(E, 1)
    ef = neighbor_edge_feature.reshape(E, D_EDGE)
    h = _attention(gk, gv, qr, fr, rts_e, nts_e, ef, wrow, wke, wkt, wve, wvt,
                   wof, woa)

    pos, neg = _predict(h[:B], h[B:2 * B], h[2 * B:], wsrc, wdst, wout)
    return jnp.concatenate([pos[:, :1], neg[:, :1]], axis=0)


# speedup vs baseline: 6.3061x; 1.5998x over previous
import functools

import jax
import jax.numpy as jnp
import numpy as np
from jax import lax
from jax.experimental import pallas as pl
from jax.experimental.pallas import tpu as pltpu
from jax.experimental.pallas import tpu_sc as plsc

N_NODES = 100000
B = 4096
N_ROOT = 3 * B
K = 32
E = N_ROOT * K
D_NODE = 128
D_EDGE = 16
PW = 128
NW = 32
E_PER_W = E // NW
EC = 128
N_ECHUNK = E_PER_W // EC
R_PER_W = N_ROOT // NW
RC = 128
N_RCHUNK = R_PER_W // RC
NB = 1000
RB = 128
SCALE = 1.0 / np.sqrt(50.0)


def _pad2(w, r, c):
    return jnp.zeros((r, c), jnp.float32).at[: w.shape[0], : w.shape[1]].set(w)


def _precompute_body(nf, mem, wkf, wkm, wvf, wvm, wqf, wqm, qb, tk, tv, tq):
    nfb = nf[...]
    memb = mem[...]
    tk[...] = jnp.dot(nfb, wkf[...], preferred_element_type=jnp.float32) + \
              jnp.dot(memb, wkm[...], preferred_element_type=jnp.float32)
    tv[...] = jnp.dot(nfb, wvf[...], preferred_element_type=jnp.float32) + \
              jnp.dot(memb, wvm[...], preferred_element_type=jnp.float32)
    tq[...] = jnp.dot(nfb, wqf[...], preferred_element_type=jnp.float32) + \
              jnp.dot(memb, wqm[...], preferred_element_type=jnp.float32) + qb[...]


def _precompute(node_feat, memory, wkf, wkm, wvf, wvm, wqf, wqm, qb):
    nblk = N_NODES // NB
    w_spec = lambda shp: pl.BlockSpec(shp, lambda i: (0, 0))
    return pl.pallas_call(
        _precompute_body,
        grid=(nblk,),
        in_specs=[
            pl.BlockSpec((NB, D_NODE), lambda i: (i, 0)),
            pl.BlockSpec((NB, 100), lambda i: (i, 0)),
            w_spec((D_NODE, PW)), w_spec((100, PW)),
            w_spec((D_NODE, PW)), w_spec((100, PW)),
            w_spec((D_NODE, PW)), w_spec((100, PW)),
            w_spec((1, PW)),
        ],
        out_specs=[
            pl.BlockSpec((NB, PW), lambda i: (i, 0)),
            pl.BlockSpec((NB, PW), lambda i: (i, 0)),
            pl.BlockSpec((NB, PW), lambda i: (i, 0)),
        ],
        out_shape=[
            jax.ShapeDtypeStruct((N_NODES, PW), jnp.float32),
            jax.ShapeDtypeStruct((N_NODES, PW), jnp.float32),
            jax.ShapeDtypeStruct((N_NODES, PW), jnp.float32),
        ],
    )(node_feat, memory, wkf, wkm, wvf, wvm, wqf, wqm, qb)


def _sc_gather(tk, tv, tq, node_feat, flat_nb, root_nid):
    mesh = plsc.VectorSubcoreMesh(core_axis_name="c", subcore_axis_name="s")

    @functools.partial(
        pl.kernel,
        mesh=mesh,
        out_type=[
            jax.ShapeDtypeStruct((E, PW), jnp.float32),
            jax.ShapeDtypeStruct((E, PW), jnp.float32),
            jax.ShapeDtypeStruct((N_ROOT, PW), jnp.float32),
            jax.ShapeDtypeStruct((N_ROOT, D_NODE), jnp.float32),
        ],
        scratch_types=[
            pltpu.VMEM((EC,), jnp.int32),
            pltpu.VMEM((EC, PW), jnp.float32),
            pltpu.VMEM((EC, PW), jnp.float32),
            pltpu.VMEM((RC,), jnp.int32),
            pltpu.VMEM((RC, PW), jnp.float32),
            pltpu.VMEM((RC, D_NODE), jnp.float32),
            pltpu.SemaphoreType.DMA,
            pltpu.SemaphoreType.DMA,
        ],
    )
    def gather_k(tk_h, tv_h, tq_h, nf_h, nbid_h, rid_h, gk_h, gv_h, qr_h, fr_h,
                 idx_v, rk_v, rv_v, idx2_v, rq_v, rf_v, sem_a, sem_b):
        wid = lax.axis_index("s") * 2 + lax.axis_index("c")
        ebase = wid * E_PER_W

        def edge_body(i, carry):
            off = ebase + i * EC
            pltpu.sync_copy(nbid_h.at[pl.ds(off, EC)], idx_v)
            cp_a = pltpu.async_copy(tk_h.at[idx_v], rk_v, sem_a)
            cp_b = pltpu.async_copy(tv_h.at[idx_v], rv_v, sem_b)
            cp_a.wait()
            cp_b.wait()
            pltpu.sync_copy(rk_v, gk_h.at[pl.ds(off, EC)])
            pltpu.sync_copy(rv_v, gv_h.at[pl.ds(off, EC)])
            return carry

        lax.fori_loop(0, N_ECHUNK, edge_body, 0)

        rbase = wid * R_PER_W

        def root_body(i, carry):
            off = rbase + i * RC
            pltpu.sync_copy(rid_h.at[pl.ds(off, RC)], idx2_v)
            cp_a = pltpu.async_copy(tq_h.at[idx2_v], rq_v, sem_a)
            cp_b = pltpu.async_copy(nf_h.at[idx2_v], rf_v, sem_b)
            cp_a.wait()
            cp_b.wait()
            pltpu.sync_copy(rq_v, qr_h.at[pl.ds(off, RC)])
            pltpu.sync_copy(rf_v, fr_h.at[pl.ds(off, RC)])
            return carry

        lax.fori_loop(0, N_RCHUNK, root_body, 0)

    return gather_k(tk, tv, tq, node_feat, flat_nb, root_nid)


_INV2PI = 0.15915494309189535
_MAGIC = 12582912.0
_C1 = 6.28125
_C2 = 0.0019353071795864769
_COSC = [1.0, -0.5, 1.0 / 24, -1.0 / 720, 1.0 / 40320, -1.0 / 3628800,
         1.0 / 479001600, -1.0 / 87178291200]


def _fast_cos(x):
    n = (x * _INV2PI + _MAGIC) - _MAGIC
    r = (x - n * _C1) - n * _C2
    r2 = r * r
    p = jnp.float32(_COSC[7])
    for c in _COSC[6::-1]:
        p = p * r2 + jnp.float32(c)
    return p


def _attn_body(gk, gv, qr, fr, rts, nts, ef, wrow, wke, wkt, wve, wvt, wof,
               woa, msel, mheads, h_out):
    f32 = jnp.float32
    et = _fast_cos((rts[...] - nts[...]) * wrow[...])
    kb = gk[...] + jnp.dot(et, wkt[...], preferred_element_type=f32) + \
         jnp.dot(ef[...], wke[...], preferred_element_type=f32)
    vb = gv[...] + jnp.dot(et, wvt[...], preferred_element_type=f32) + \
         jnp.dot(ef[...], wve[...], preferred_element_type=f32)
    q = qr[...]
    p3 = kb.reshape(RB, K, PW) * q[:, None, :]
    s8 = jnp.dot(p3.reshape(RB * K, PW), msel[...],
                 preferred_element_type=f32)
    s3 = jnp.clip(s8.reshape(RB, K, 8) * SCALE, -10.0, 10.0)
    e3 = jnp.exp(s3 - jnp.max(s3, axis=1, keepdims=True))
    a3 = e3 / jnp.sum(e3, axis=1, keepdims=True)
    w2 = jnp.dot(a3.reshape(RB * K, 8), mheads[...],
                 preferred_element_type=f32)
    agg = jnp.sum((w2 * vb).reshape(RB, K, PW), axis=1)
    h = jnp.maximum(
        jnp.dot(fr[...], wof[...], preferred_element_type=f32) +
        jnp.dot(agg, woa[...], preferred_element_type=f32), 0.0)
    h_out[...] = h


def _attention(gk, gv, qr, fr, rts_e, nts_e, ef, wrow, wke, wkt, wve, wvt,
               wof, woa, msel, mheads):
    nblk = N_ROOT // RB
    w_spec = lambda shp: pl.BlockSpec(shp, lambda i: (0, 0))
    return pl.pallas_call(
        _attn_body,
        grid=(nblk,),
        in_specs=[
            pl.BlockSpec((RB * K, PW), lambda i: (i, 0)),
            pl.BlockSpec((RB * K, PW), lambda i: (i, 0)),
            pl.BlockSpec((RB, PW), lambda i: (i, 0)),
            pl.BlockSpec((RB, D_NODE), lambda i: (i, 0)),
            pl.BlockSpec((RB * K, 1), lambda i: (i, 0)),
            pl.BlockSpec((RB * K, 1), lambda i: (i, 0)),
            pl.BlockSpec((RB * K, D_EDGE), lambda i: (i, 0)),
            w_spec((1, PW)),
            w_spec((D_EDGE, PW)), w_spec((PW, PW)),
            w_spec((D_EDGE, PW)), w_spec((PW, PW)),
            w_spec((D_NODE, PW)), w_spec((PW, PW)),
            w_spec((PW, 8)), w_spec((8, PW)),
        ],
        out_specs=pl.BlockSpec((RB, PW), lambda i: (i, 0)),
        out_shape=jax.ShapeDtypeStruct((N_ROOT, PW), jnp.float32),
    )(gk, gv, qr, fr, rts_e, nts_e, ef, wrow, wke, wkt, wve, wvt, wof, woa,
      msel, mheads)


def _pred_body(hs, hp, hn, wsrc, wdst, wout, pos, neg):
    f32 = jnp.float32
    h_src = jnp.dot(hs[...], wsrc[...], preferred_element_type=f32)
    h_pos = jnp.dot(hp[...], wdst[...], preferred_element_type=f32)
    h_neg = jnp.dot(hn[...], wdst[...], preferred_element_type=f32)
    pos[...] = jnp.dot(jnp.maximum(h_src + h_pos, 0.0), wout[...],
                       preferred_element_type=f32)
    neg[...] = jnp.dot(jnp.maximum(h_src + h_neg, 0.0), wout[...],
                       preferred_element_type=f32)


def _predict(h_src, h_pos, h_neg, wsrc, wdst, wout):
    PB = 512
    nblk = B // PB
    w_spec = lambda shp: pl.BlockSpec(shp, lambda i: (0, 0))
    return pl.pallas_call(
        _pred_body,
        grid=(nblk,),
        in_specs=[
            pl.BlockSpec((PB, PW), lambda i: (i, 0)),
            pl.BlockSpec((PB, PW), lambda i: (i, 0)),
            pl.BlockSpec((PB, PW), lambda i: (i, 0)),
            w_spec((PW, PW)), w_spec((PW, PW)), w_spec((PW, 8)),
        ],
        out_specs=[
            pl.BlockSpec((PB, 8), lambda i: (i, 0)),
            pl.BlockSpec((PB, 8), lambda i: (i, 0)),
        ],
        out_shape=[
            jax.ShapeDtypeStruct((B, 8), jnp.float32),
            jax.ShapeDtypeStruct((B, 8), jnp.float32),
        ],
    )(h_src, h_pos, h_neg, wsrc, wdst, wout)


def kernel(node_feat, memory, root_nid, neighbor_nid, root_ts, neighbor_ts,
           neighbor_edge_feature, Wq, Wk, Wv, Wo, Wsrc, Wdst, Wout):
    f32 = jnp.float32
    wqf = _pad2(Wq[:128], D_NODE, PW)
    wqm = _pad2(Wq[228:], 100, PW)
    qb = _pad2(jnp.sum(Wq[128:228], 0, keepdims=True), 1, PW)
    wkf = _pad2(Wk[:128], D_NODE, PW)
    wke = _pad2(Wk[128:144], D_EDGE, PW)
    wkt = _pad2(Wk[144:244], PW, PW)
    wkm = _pad2(Wk[244:], 100, PW)
    wvf = _pad2(Wv[:128], D_NODE, PW)
    wve = _pad2(Wv[128:144], D_EDGE, PW)
    wvt = _pad2(Wv[144:244], PW, PW)
    wvm = _pad2(Wv[244:], 100, PW)
    wof = _pad2(Wo[:128], D_NODE, PW)
    woa = _pad2(Wo[128:], PW, PW)
    wsrc = _pad2(Wsrc, PW, PW)
    wdst = _pad2(Wdst, PW, PW)
    wout = _pad2(Wout, PW, 8)
    wrow = _pad2((1.0 / (10.0 ** jnp.linspace(0.0, 9.0, 100, dtype=f32)))[None, :], 1, PW)
    lanes = np.arange(PW)
    msel_np = np.zeros((PW, 8), np.float32)
    msel_np[lanes < 50, 0] = 1.0
    msel_np[(lanes >= 50) & (lanes < 100), 1] = 1.0
    msel = jnp.asarray(msel_np)
    mheads = jnp.asarray(msel_np.T.copy())

    tk, tv, tq = _precompute(node_feat, memory, wkf, wkm, wvf, wvm, wqf, wqm, qb)

    flat_nb = neighbor_nid.reshape(-1).astype(jnp.int32)
    rid = root_nid.astype(jnp.int32)
    gk, gv, qr, fr = _sc_gather(tk, tv, tq, node_feat, flat_nb, rid)

    rts_e = jnp.broadcast_to(root_ts[:, None], (N_ROOT, K)).reshape(E, 1)
    nts_e = neighbor_ts.reshape(E, 1)
    ef = neighbor_edge_feature.reshape(E, D_EDGE)
    h = _attention(gk, gv, qr, fr, rts_e, nts_e, ef, wrow, wke, wkt, wve, wvt,
                   wof, woa, msel, mheads)

    pos, neg = _predict(h[:B], h[B:2 * B], h[2 * B:], wsrc, wdst, wout)
    return jnp.concatenate([pos[:, :1], neg[:, :1]], axis=0)
